# Initial kernel scaffold; baseline (speedup 1.0000x reference)
import jax
import jax.numpy as jnp
from jax.experimental import pallas as pl


def kernel(ints, byte_table, W, b):
    raise NotImplementedError("write your pallas kernel here")


# speedup vs baseline: 2.2743x; 2.2743x over previous
import functools

import jax
import jax.numpy as jnp
from jax import lax
from jax.experimental import pallas as pl
from jax.experimental.pallas import tpu as pltpu
from jax.experimental.pallas import tpu_sc as plsc

BYTES = 4
ED = 128
EDB = 32
B = 16384
NC, NS = 2, 16
NW = NC * NS
BPW = B // NW


def _sc_gather(ints, tbl_flat):
    mesh = plsc.VectorSubcoreMesh(
        core_axis_name="c", subcore_axis_name="s", num_cores=NC, num_subcores=NS
    )

    @functools.partial(
        pl.kernel,
        out_type=jax.ShapeDtypeStruct((B * ED,), jnp.float32),
        mesh=mesh,
        compiler_params=pltpu.CompilerParams(needs_layout_passes=False),
        scratch_types=[
            pltpu.VMEM((BPW,), jnp.int32),
            pltpu.VMEM((256 * EDB,), jnp.float32),
            pltpu.VMEM((BPW * ED,), jnp.float32),
        ],
    )
    def body(ints_hbm, tbl_hbm, out_hbm, ints_v, tbl_v, out_v):
        wid = lax.axis_index("s") * NC + lax.axis_index("c")
        base = wid * BPW
        pltpu.sync_copy(tbl_hbm, tbl_v)
        pltpu.sync_copy(ints_hbm.at[pl.ds(base, BPW)], ints_v)
        lanes = lax.iota(jnp.int32, 16)

        def group(g, carry):
            v = ints_v[pl.ds(g * 16, 16)]
            out_base = (lanes + g * 16) * ED
            for j in range(BYTES):
                idx = ((v >> (8 * (BYTES - 1 - j))) & 0xFF) * EDB
                for c in range(EDB):
                    col = plsc.load_gather(tbl_v, [idx + c])
                    plsc.store_scatter(out_v, [out_base + (j * EDB + c)], col)
            return carry

        lax.fori_loop(0, BPW // 16, group, 0)
        pltpu.sync_copy(out_v, out_hbm.at[pl.ds(base * ED, BPW * ED)])

    return body(ints, tbl_flat)


def _mm_body(x_ref, w_ref, b_ref, o_ref):
    o_ref[...] = (
        lax.dot_general(
            x_ref[...], w_ref[...], (((1,), (1,)), ((), ())),
            preferred_element_type=jnp.float32,
        )
        + b_ref[...]
    )


def _tc_matmul(flat, W, b):
    blk = 1024
    return pl.pallas_call(
        _mm_body,
        out_shape=jax.ShapeDtypeStruct((B, ED), jnp.float32),
        grid=(B // blk,),
        in_specs=[
            pl.BlockSpec((blk, ED), lambda i: (i, 0)),
            pl.BlockSpec((ED, ED), lambda i: (0, 0)),
            pl.BlockSpec((1, ED), lambda i: (0, 0)),
        ],
        out_specs=pl.BlockSpec((blk, ED), lambda i: (i, 0)),
    )(flat, W, b.reshape(1, ED))


def kernel(ints, byte_table, W, b):
    flat = _sc_gather(ints, byte_table.reshape(-1))
    return _tc_matmul(flat.reshape(B, ED), W, b)


# speedup vs baseline: 5.1862x; 2.2803x over previous
import functools

import jax
import jax.numpy as jnp
from jax import lax
from jax.experimental import pallas as pl
from jax.experimental.pallas import tpu as pltpu
from jax.experimental.pallas import tpu_sc as plsc

BYTES = 4
ED = 128
EDB = 32
B = 16384
NC, NS = 2, 16
NW = NC * NS
BPW = B // NW


def _sc_gather(ints, tbl_flat):
    mesh = plsc.VectorSubcoreMesh(
        core_axis_name="c", subcore_axis_name="s", num_cores=NC, num_subcores=NS
    )

    @functools.partial(
        pl.kernel,
        out_type=jax.ShapeDtypeStruct((B * ED,), jnp.float32),
        mesh=mesh,
        compiler_params=pltpu.CompilerParams(needs_layout_passes=False),
        scratch_types=[
            pltpu.VMEM((BPW,), jnp.int32),
            pltpu.VMEM((256 * EDB,), jnp.float32),
            pltpu.VMEM((BPW * ED,), jnp.float32),
        ],
    )
    def body(ints_hbm, tbl_hbm, out_hbm, ints_v, tbl_v, out_v):
        wid = lax.axis_index("s") * NC + lax.axis_index("c")
        base = wid * BPW
        pltpu.sync_copy(tbl_hbm, tbl_v)
        pltpu.sync_copy(ints_hbm.at[pl.ds(base, BPW)], ints_v)
        lanes = lax.iota(jnp.int32, 16)

        def group(g, carry):
            v = ints_v[pl.ds(g * 16, 16)]
            out_base = (lanes + g * 16) * ED
            idx = [
                ((v >> (8 * (BYTES - 1 - j))) & 0xFF) * EDB
                for j in range(BYTES)
            ]
            for c0 in range(EDB):
                coff = (lanes + c0) & (EDB - 1)
                obc = out_base + coff
                vals = [plsc.load_gather(tbl_v, [idx[j] + coff]) for j in range(BYTES)]
                for j in range(BYTES):
                    plsc.store_scatter(out_v, [obc + j * EDB], vals[j])
            return carry

        lax.fori_loop(0, BPW // 16, group, 0)
        pltpu.sync_copy(out_v, out_hbm.at[pl.ds(base * ED, BPW * ED)])

    return body(ints, tbl_flat)


def _mm_body(x_ref, w_ref, b_ref, o_ref):
    o_ref[...] = (
        lax.dot_general(
            x_ref[...], w_ref[...], (((1,), (1,)), ((), ())),
            preferred_element_type=jnp.float32,
        )
        + b_ref[...]
    )


def _tc_matmul(flat, W, b):
    blk = 1024
    return pl.pallas_call(
        _mm_body,
        out_shape=jax.ShapeDtypeStruct((B, ED), jnp.float32),
        grid=(B // blk,),
        in_specs=[
            pl.BlockSpec((blk, ED), lambda i: (i, 0)),
            pl.BlockSpec((ED, ED), lambda i: (0, 0)),
            pl.BlockSpec((1, ED), lambda i: (0, 0)),
        ],
        out_specs=pl.BlockSpec((blk, ED), lambda i: (i, 0)),
    )(flat, W, b.reshape(1, ED))


def kernel(ints, byte_table, W, b):
    flat = _sc_gather(ints, byte_table.reshape(-1))
    return _tc_matmul(flat.reshape(B, ED), W, b)


# speedup vs baseline: 6.0831x; 1.1729x over previous
import functools

import jax
import jax.numpy as jnp
from jax import lax
from jax.experimental import pallas as pl
from jax.experimental.pallas import tpu as pltpu
from jax.experimental.pallas import tpu_sc as plsc

BYTES = 4
ED = 128
EDB = 32
B = 16384
NC, NS = 2, 16
NW = NC * NS
BPW = B // NW
CH = 128
TW = 1024 * (ED // 2)


def _fuse_pack_body(bt_ref, w_ref, b_ref, o_ref):
    bt = bt_ref[...]
    w = w_ref[...]
    for j in range(BYTES):
        wj = w[:, EDB * j:EDB * (j + 1)]
        t = lax.dot_general(
            bt, wj, (((1,), (1,)), ((), ())), preferred_element_type=jnp.float32
        )
        if j == 0:
            t = t + b_ref[...]
        lo = lax.bitcast_convert_type(
            t[:, : ED // 2].astype(jnp.bfloat16), jnp.uint16
        ).astype(jnp.uint32)
        hi = lax.bitcast_convert_type(
            t[:, ED // 2:].astype(jnp.bfloat16), jnp.uint16
        ).astype(jnp.uint32)
        o_ref[256 * j:256 * (j + 1), :] = (hi << 16) | lo


def _fuse_pack(byte_table, W, b):
    return pl.pallas_call(
        _fuse_pack_body,
        out_shape=jax.ShapeDtypeStruct((1024, ED // 2), jnp.uint32),
    )(byte_table, W, b.reshape(1, ED))


def _sc_embed(ints, tbl_packed):
    mesh = plsc.VectorSubcoreMesh(
        core_axis_name="c", subcore_axis_name="s", num_cores=NC, num_subcores=NS
    )

    @functools.partial(
        pl.kernel,
        out_type=jax.ShapeDtypeStruct((B, ED), jnp.float32),
        mesh=mesh,
        compiler_params=pltpu.CompilerParams(needs_layout_passes=False),
        scratch_types=[
            pltpu.VMEM((BPW,), jnp.int32),
            pltpu.VMEM((TW,), jnp.uint32),
            pltpu.VMEM((CH, ED), jnp.float32),
            pltpu.VMEM((CH, ED), jnp.float32),
            pltpu.SemaphoreType.DMA,
            pltpu.SemaphoreType.DMA,
        ],
    )
    def body(ints_hbm, tbl_hbm, out_hbm, ints_v, tbl_v, out_a, out_b,
             sem_a, sem_b):
        wid = lax.axis_index("s") * NC + lax.axis_index("c")
        base = wid * BPW
        pltpu.sync_copy(tbl_hbm, tbl_v)
        pltpu.sync_copy(ints_hbm.at[pl.ds(base, BPW)], ints_v)
        mask_hi = jnp.uint32(0xFFFF0000)
        bufs = (out_a, out_b)
        sems = (sem_a, sem_b)

        def make_chunk(buf, off):
            def one(g, carry):
                v = ints_v[pl.ds(off + g * 16, 16)]
                for l in range(16):
                    s = v[l]
                    i = g * 16 + l
                    acc_lo = [None] * (ED // 32)
                    acc_hi = [None] * (ED // 32)
                    for j in range(BYTES):
                        r = ((s >> (8 * (BYTES - 1 - j))) & 0xFF) + 256 * j
                        rb = r * (ED // 2)
                        for k in range(ED // 32):
                            u = tbl_v[pl.ds(rb + 16 * k, 16)]
                            lo = plsc.bitcast(u << 16, jnp.float32)
                            hi = plsc.bitcast(u & mask_hi, jnp.float32)
                            if j == 0:
                                acc_lo[k] = lo
                                acc_hi[k] = hi
                            else:
                                acc_lo[k] = acc_lo[k] + lo
                                acc_hi[k] = acc_hi[k] + hi
                    for k in range(ED // 32):
                        buf[i, pl.ds(16 * k, 16)] = acc_lo[k]
                        buf[i, pl.ds(ED // 2 + 16 * k, 16)] = acc_hi[k]
                return carry

            return one

        pending = [None, None]
        for c in range(BPW // CH):
            p = c % 2
            if pending[p] is not None:
                pending[p].wait()
            lax.fori_loop(0, CH // 16, make_chunk(bufs[p], c * CH), 0)
            pending[p] = pltpu.async_copy(
                bufs[p], out_hbm.at[pl.ds(base + c * CH, CH)], sems[p]
            )
        for p in range(2):
            if pending[p] is not None:
                pending[p].wait()

    return body(ints, tbl_packed)


def kernel(ints, byte_table, W, b):
    tbl = _fuse_pack(byte_table, W, b).reshape(-1)
    return _sc_embed(ints, tbl)
